# Initial kernel scaffold; baseline (speedup 1.0000x reference)
import jax
import jax.numpy as jnp
from jax.experimental import pallas as pl


def kernel(all_embs, inputs):
    raise NotImplementedError("write your pallas kernel here")


# speedup vs baseline: 1.1091x; 1.1091x over previous
import functools

import jax
import jax.numpy as jnp
from jax import lax
from jax.experimental import pallas as pl
from jax.experimental.pallas import tpu as pltpu
from jax.experimental.pallas import tpu_sc as plsc

NC = 2
NS = 16
NW = NC * NS

CHUNK = 128
NBUF = 4


def _gather_kernel(n_per_w, n_chunks, dim, table_hbm, idx_hbm, out_hbm,
                   idx_v, rows_v, gsems):
    wid = lax.axis_index("s") * NC + lax.axis_index("c")
    row0 = wid * n_chunks

    pltpu.sync_copy(idx_hbm.at[pl.ds(row0, n_chunks)], idx_v)

    def start_gather(j, b):
        pltpu.async_copy(table_hbm.at[idx_v.at[j]], rows_v.at[b], gsems.at[b])

    for b in range(NBUF):
        start_gather(b, b)

    n_groups = n_chunks // NBUF

    def body(g, _):
        for b in range(NBUF):
            j = g * NBUF + b
            pltpu.make_async_copy(table_hbm.at[idx_v.at[j]], rows_v.at[b],
                                  gsems.at[b]).wait()
            base = (row0 + j) * CHUNK
            pltpu.sync_copy(rows_v.at[b], out_hbm.at[pl.ds(base, CHUNK)])
            @pl.when(g < n_groups - 1)
            def _():
                start_gather(j + NBUF, b)
        return ()

    lax.fori_loop(0, n_groups, body, (), unroll=False)


def kernel(all_embs, inputs):
    V, D = all_embs.shape
    B, L = inputs.shape
    N = B * L
    assert N % (NW * CHUNK * NBUF) == 0
    n_per_w = N // NW
    n_chunks = n_per_w // CHUNK

    idx_2d = inputs.reshape(N // CHUNK, CHUNK)

    mesh = plsc.VectorSubcoreMesh(core_axis_name="c", subcore_axis_name="s")
    out = pl.kernel(
        functools.partial(_gather_kernel, n_per_w, n_chunks, D),
        out_type=jax.ShapeDtypeStruct((N, D), jnp.float32),
        mesh=mesh,
        scratch_types=[
            pltpu.VMEM((n_chunks, CHUNK), jnp.int32),
            pltpu.VMEM((NBUF, CHUNK, D), jnp.float32),
            pltpu.SemaphoreType.DMA((NBUF,)),
        ],
        compiler_params=pltpu.CompilerParams(use_tc_tiling_on_sc=False),
    )(all_embs, idx_2d)
    return out.reshape(B, L, D)


# speedup vs baseline: 1.1115x; 1.0022x over previous
import functools

import jax
import jax.numpy as jnp
from jax import lax
from jax.experimental import pallas as pl
from jax.experimental.pallas import tpu as pltpu
from jax.experimental.pallas import tpu_sc as plsc

NC = 2
NS = 16
NW = NC * NS

CHUNK = 128
DEPTH = 4
M = 2 * DEPTH


def _gather_kernel(n_chunks, dim, table_hbm, idx_hbm, out_hbm,
                   idx_v, rows_v, gsems, wsems):
    wid = lax.axis_index("s") * NC + lax.axis_index("c")
    row0 = wid * n_chunks

    pltpu.sync_copy(idx_hbm.at[pl.ds(row0, n_chunks)], idx_v)

    def gather(j, b):
        return pltpu.make_async_copy(table_hbm.at[idx_v.at[j]],
                                     rows_v.at[b], gsems.at[b])

    def write(j, b):
        return pltpu.make_async_copy(rows_v.at[b],
                                     out_hbm.at[pl.ds((row0 + j) * CHUNK, CHUNK)],
                                     wsems.at[b])

    for b in range(DEPTH):
        gather(b, b).start()

    n_groups = n_chunks // M

    def body(g, _):
        for b in range(M):
            j = g * M + b
            gather(j, b).wait()
            write(j, b).start()
            bn = (b + DEPTH) % M
            jn = j + DEPTH

            @pl.when(jn < n_chunks)
            def _():
                @pl.when(jn >= M)
                def _():
                    write(jn - M, bn).wait()
                gather(jn, bn).start()
        return ()

    lax.fori_loop(0, n_groups, body, (), unroll=False)

    for b in range(M):
        j = n_chunks - M + b
        write(j, b).wait()


def kernel(all_embs, inputs):
    V, D = all_embs.shape
    B, L = inputs.shape
    N = B * L
    n_per_w = N // NW
    n_chunks = n_per_w // CHUNK
    assert N % (NW * CHUNK) == 0 and n_chunks % M == 0

    idx_2d = inputs.reshape(N // CHUNK, CHUNK)

    mesh = plsc.VectorSubcoreMesh(core_axis_name="c", subcore_axis_name="s")
    out = pl.kernel(
        functools.partial(_gather_kernel, n_chunks, D),
        out_type=jax.ShapeDtypeStruct((N, D), jnp.float32),
        mesh=mesh,
        scratch_types=[
            pltpu.VMEM((n_chunks, CHUNK), jnp.int32),
            pltpu.VMEM((M, CHUNK, D), jnp.float32),
            pltpu.SemaphoreType.DMA((M,)),
            pltpu.SemaphoreType.DMA((M,)),
        ],
        compiler_params=pltpu.CompilerParams(use_tc_tiling_on_sc=False),
    )(all_embs, idx_2d)
    return out.reshape(B, L, D)


# speedup vs baseline: 1.6241x; 1.4612x over previous
import functools

import jax
import jax.numpy as jnp
from jax import lax
from jax.experimental import pallas as pl
from jax.experimental.pallas import tpu as pltpu
from jax.experimental.pallas import tpu_sc as plsc

NC = 2
NS = 16
NW = NC * NS


def _lookup_kernel(n_bt, L, D, table_hbm, idx_hbm, out_hbm,
                   idxb, idxt, rows, outst, gsems, wsems):
    wid = lax.axis_index("s") * NC + lax.axis_index("c")
    nct = D // 8
    bt_per_w = n_bt // NW

    iota = lax.iota(jnp.int32, 16)
    iota_l = iota * L
    iota_d = iota * D

    def gather(l, b):
        return pltpu.make_async_copy(
            table_hbm.at[idxt.at[pl.ds(l * 128, 128)]], rows.at[b], gsems.at[b])

    def write(l, nt, b, ct):
        return pltpu.make_async_copy(
            outst.at[b, pl.ds(ct * 1024, 1024)], out_hbm.at[l, ct, nt],
            wsems.at[b])

    for t in range(bt_per_w):
        nt = wid * bt_per_w + t
        pltpu.sync_copy(idx_hbm.at[pl.ds(nt * 128 * L, 128 * L)], idxb)

        def tr_idx(l, _):
            for g in range(8):
                v = plsc.load_gather(idxb, [iota_l + (g * 16 * L + l)])
                idxt[pl.ds(l * 128 + g * 16, 16)] = v
            return ()
        lax.fori_loop(0, L, tr_idx, (), unroll=False)

        gather(0, 0).start()

        def body(h, _):
            for b in range(2):
                l = h * 2 + b
                gather(l, b).wait()

                @pl.when(l + 1 < L)
                def _():
                    gather(l + 1, 1 - b).start()

                @pl.when(l >= 2)
                def _():
                    for ct in range(nct):
                        write(l - 2, nt, b, ct).wait()

                def tr_rows(c, _):
                    j = jnp.broadcast_to(c, (16,))
                    for g in range(8):
                        v = plsc.load_gather(rows.at[b], [iota + g * 16, j])
                        outst[b, pl.ds(c * 128 + g * 16, 16)] = v
                    return ()
                lax.fori_loop(0, D, tr_rows, (), unroll=False)

                for ct in range(nct):
                    write(l, nt, b, ct).start()
            return ()

        lax.fori_loop(0, L // 2, body, (), unroll=False)

        for b in range(2):
            for ct in range(nct):
                write(L - 2 + b, nt, b, ct).wait()


def kernel(all_embs, inputs):
    V, D = all_embs.shape
    B, L = inputs.shape
    n_bt = B // 128
    assert B % 128 == 0 and n_bt % NW == 0 and D % 8 == 0 and L % 2 == 0

    idx_flat = inputs.reshape(B * L)

    mesh = plsc.VectorSubcoreMesh(core_axis_name="c", subcore_axis_name="s")
    out4 = pl.kernel(
        functools.partial(_lookup_kernel, n_bt, L, D),
        out_type=jax.ShapeDtypeStruct((L, D // 8, n_bt, 1024), jnp.float32),
        mesh=mesh,
        scratch_types=[
            pltpu.VMEM((128 * L,), jnp.int32),
            pltpu.VMEM((L * 128,), jnp.int32),
            pltpu.VMEM((2, 128, D), jnp.float32),
            pltpu.VMEM((2, (D // 8) * 1024), jnp.float32),
            pltpu.SemaphoreType.DMA((2,)),
            pltpu.SemaphoreType.DMA((2,)),
        ],
        compiler_params=pltpu.CompilerParams(use_tc_tiling_on_sc=False,
                                             needs_layout_passes=False),
    )(all_embs, idx_flat)

    out = out4.reshape(L, D // 8, n_bt, 8, 128)
    out = out.transpose(2, 4, 0, 1, 3)
    return out.reshape(B, L, D)
